# baseline (device time: 105097 ns/iter reference)
import jax
import jax.numpy as jnp
from jax import lax
from jax.experimental import pallas as pl
from jax.experimental.pallas import tpu as pltpu

N = 16
B, S, D = 2, 512, 768
R = B * S
CH = R // N
CPB = N // B
HQ = 4
DH = 96
SCALE = 0.10206207261596577
EPS = 1e-5
WIRE = jnp.bfloat16


def kernel(x, Wq, Wk, Wv, Wo, t_emb, W_mod, W_ff1, W_ff2):
    def body(x_ref, wq_ref, wk_ref, wv_ref, wo_ref, temb_ref, wmod_ref,
             wff1_ref, wff2_ref, out_ref,
             pbuf, a2a_buf, bc_buf, p2buf, a2a2_buf, bc2_buf,
             attn_ref, x1_ref,
             sc_send_sems, sc_recv_sems, bc_send_sems, bc_recv_sems):
        my = lax.axis_index("i")
        bf = jnp.bfloat16
        sends = []

        barrier = pltpu.get_barrier_semaphore()
        for off in range(1, N):
            pl.semaphore_signal(barrier, inc=1,
                                device_id=(lax.rem(my + off, N),),
                                device_id_type=pl.DeviceIdType.MESH)
        pl.semaphore_wait(barrier, N - 1)

        def scatter_send(src_buf, dst_buf, j, ph):
            rdma = pltpu.make_async_remote_copy(
                src_ref=src_buf.at[j],
                dst_ref=dst_buf.at[my],
                send_sem=sc_send_sems.at[ph, j],
                recv_sem=sc_recv_sems.at[ph, my],
                device_id=(j,),
                device_id_type=pl.DeviceIdType.MESH,
            )
            @pl.when(my != j)
            def _():
                rdma.start()
            sends.append((rdma, j))

        def scatter_wait(dst_buf, src, ph):
            rdma = pltpu.make_async_remote_copy(
                src_ref=dst_buf.at[src],
                dst_ref=dst_buf.at[src],
                send_sem=sc_send_sems.at[ph, src],
                recv_sem=sc_recv_sems.at[ph, src],
                device_id=(src,),
                device_id_type=pl.DeviceIdType.MESH,
            )
            @pl.when(my != src)
            def _():
                rdma.wait_recv()

        def bcast_send(buf, j, ph):
            rdma = pltpu.make_async_remote_copy(
                src_ref=buf.at[my],
                dst_ref=buf.at[my],
                send_sem=bc_send_sems.at[ph, j],
                recv_sem=bc_recv_sems.at[ph, my],
                device_id=(j,),
                device_id_type=pl.DeviceIdType.MESH,
            )
            @pl.when(my != j)
            def _():
                rdma.start()
            sends.append((rdma, j))

        def bcast_wait(buf, j, ph):
            rdma = pltpu.make_async_remote_copy(
                src_ref=buf.at[j],
                dst_ref=buf.at[j],
                send_sem=bc_send_sems.at[ph, j],
                recv_sem=bc_recv_sems.at[ph, j],
                device_id=(j,),
                device_id_type=pl.DeviceIdType.MESH,
            )
            @pl.when(my != j)
            def _():
                rdma.wait_recv()

        def ln(h):
            m = jnp.mean(h, axis=-1, keepdims=True)
            c = h - m
            v = jnp.mean(c * c, axis=-1, keepdims=True)
            return c * lax.rsqrt(v + EPS)

        mod = jnp.dot(temb_ref[...], wmod_ref[...],
                      preferred_element_type=jnp.float32)
        sa, sha, ga, sm_, shm, gm = [mod[:, i * D:(i + 1) * D] for i in range(6)]

        x0 = jnp.reshape(x_ref[...], (R, D))
        xa3 = jnp.reshape(ln(x0), (B, S, D))
        xa3 = (xa3 * (1.0 + sa[:, None, :]) + sha[:, None, :]).astype(bf)

        wo_bf = wo_ref[...].astype(bf)
        for b in range(B):
            xab = xa3[b]
            q = jnp.dot(xab, wq_ref[...].astype(bf), preferred_element_type=jnp.float32)
            k = jnp.dot(xab, wk_ref[...].astype(bf), preferred_element_type=jnp.float32)
            v = jnp.dot(xab, wv_ref[...].astype(bf), preferred_element_type=jnp.float32)
            for h in range(HQ):
                qb = q[:, h * DH:(h + 1) * DH].astype(bf)
                kb = k[:, h * DH:(h + 1) * DH].astype(bf)
                vb = v[:, h * DH:(h + 1) * DH].astype(bf)
                s_ = lax.dot_general(qb, kb, (((1,), (1,)), ((), ())),
                                     preferred_element_type=jnp.float32) * SCALE
                mx = jnp.max(s_, axis=-1, keepdims=True)
                p = jnp.exp(s_ - mx)
                l = jnp.sum(p, axis=-1, keepdims=True)
                o = jnp.dot(p.astype(bf), vb,
                            preferred_element_type=jnp.float32) / l
                attn_ref[b * S:(b + 1) * S, h * DH:(h + 1) * DH] = o
            p1b = jnp.dot(attn_ref[b * S:(b + 1) * S, :].astype(bf), wo_bf,
                          preferred_element_type=jnp.float32)
            pbuf[b * CPB:(b + 1) * CPB] = jnp.reshape(p1b.astype(WIRE),
                                                      (CPB, CH, D))
            for j in range(b * CPB, (b + 1) * CPB):
                scatter_send(pbuf, a2a_buf, j, 0)

        a2a_buf[my, :, :] = pbuf[my]
        for src in range(N):
            scatter_wait(a2a_buf, src, 0)
        red = jnp.sum(a2a_buf[...].astype(jnp.float32), axis=0)
        bc_buf[my, :, :] = red.astype(WIRE)
        for j in range(N):
            bcast_send(bc_buf, j, 0)

        wf1_bf = wff1_ref[...].astype(bf)
        wf2_bf = wff2_ref[...].astype(bf)
        for j in range(N):
            bcast_wait(bc_buf, j, 0)
            b = j // CPB
            r0 = j * CH
            x1c = x0[r0:r0 + CH, :] + ga[b][None, :] * bc_buf[j].astype(jnp.float32)
            x1_ref[r0:r0 + CH, :] = x1c
            xmc = (ln(x1c) * (1.0 + sm_[b][None, :]) + shm[b][None, :]).astype(bf)
            hf = jnp.dot(xmc, wf1_bf, preferred_element_type=jnp.float32)
            hf = hf * (1.0 / (1.0 + jnp.exp(-hf)))
            p2c = jnp.dot(hf.astype(bf), wf2_bf,
                          preferred_element_type=jnp.float32)
            p2buf[j, :, :] = p2c.astype(WIRE)
            scatter_send(p2buf, a2a2_buf, j, 1)

        a2a2_buf[my, :, :] = p2buf[my]
        for src in range(N):
            scatter_wait(a2a2_buf, src, 1)
        red2 = jnp.sum(a2a2_buf[...].astype(jnp.float32), axis=0)
        bc2_buf[my, :, :] = red2.astype(WIRE)
        for j in range(N):
            bcast_send(bc2_buf, j, 1)

        for j in range(N):
            bcast_wait(bc2_buf, j, 1)
            b = j // CPB
            s0 = (j % CPB) * CH
            outc = (x1_ref[j * CH:(j + 1) * CH, :]
                    + gm[b][None, :] * bc2_buf[j].astype(jnp.float32))
            out_ref[b, s0:s0 + CH, :] = outc

        for r, j in sends:
            @pl.when(my != j)
            def _():
                r.wait_send()

    def body_wrapped(*refs):
        body(*refs)

    return pl.pallas_call(
        body_wrapped,
        out_shape=jax.ShapeDtypeStruct((B, S, D), jnp.float32),
        in_specs=[pl.BlockSpec(memory_space=pltpu.VMEM)] * 9,
        out_specs=pl.BlockSpec(memory_space=pltpu.VMEM),
        scratch_shapes=[
            pltpu.VMEM((N, CH, D), WIRE),
            pltpu.VMEM((N, CH, D), WIRE),
            pltpu.VMEM((N, CH, D), WIRE),
            pltpu.VMEM((N, CH, D), WIRE),
            pltpu.VMEM((N, CH, D), WIRE),
            pltpu.VMEM((N, CH, D), WIRE),
            pltpu.VMEM((R, HQ * DH), jnp.float32),
            pltpu.VMEM((R, D), jnp.float32),
            pltpu.SemaphoreType.DMA((2, N)),
            pltpu.SemaphoreType.DMA((2, N)),
            pltpu.SemaphoreType.DMA((2, N)),
            pltpu.SemaphoreType.DMA((2, N)),
        ],
        compiler_params=pltpu.CompilerParams(collective_id=0),
    )(x, Wq, Wk, Wv, Wo, t_emb, W_mod, W_ff1, W_ff2)
